# baseline (speedup 1.0000x reference)
import functools

import jax
import jax.numpy as jnp
from jax import lax
from jax.experimental import pallas as pl
from jax.experimental.pallas import tpu as pltpu
from jax.experimental.pallas import tpu_sc as plsc

_LANES = 16


def _f32_key(f):
    u = plsc.bitcast(f, jnp.uint32)
    return jnp.where(f < 0.0, ~u, u | jnp.uint32(0x80000000))


def _sc_mask_body(keep, d_total, n_workers, n_cores,
                  imp_hbm, mask_hbm, imp_v, key_v, out_v):
    per = d_total // n_workers
    ngrp = per // _LANES
    nvec = d_total // _LANES
    wid = lax.axis_index("s") * n_cores + lax.axis_index("c")
    base = wid * per
    pltpu.sync_copy(imp_hbm, imp_v)
    iota = lax.broadcasted_iota(jnp.int32, (_LANES,), 0)

    for j in range(nvec):
        key_v[pl.ds(j * _LANES, _LANES)] = _f32_key(
            imp_v[pl.ds(j * _LANES, _LANES)])

    nacc = 8

    def bit_body(i, thr):
        t = thr | (jnp.uint32(1) << (jnp.uint32(31) - i.astype(jnp.uint32)))
        tv = jnp.broadcast_to(t, (_LANES,))
        accs = [jnp.zeros((_LANES,), jnp.int32) for _ in range(nacc)]
        for j in range(nvec):
            accs[j % nacc] = accs[j % nacc] + (
                key_v[pl.ds(j * _LANES, _LANES)] >= tv).astype(jnp.int32)
        while len(accs) > 1:
            accs = [a + b for a, b in zip(accs[::2], accs[1::2])]
        c = jnp.sum(accs[0])
        return jnp.where(c >= keep, t, thr)

    thr = lax.fori_loop(0, 32, bit_body, jnp.uint32(0))
    thrv = jnp.broadcast_to(thr, (_LANES,))

    accs_gt = [jnp.zeros((_LANES,), jnp.int32) for _ in range(nacc)]
    acc_pf = [jnp.zeros((_LANES,), jnp.int32) for _ in range(ngrp)]
    for j in range(nvec):
        kv = key_v[pl.ds(j * _LANES, _LANES)]
        accs_gt[j % nacc] = accs_gt[j % nacc] + (
            kv > thrv).astype(jnp.int32)
        eq = (kv == thrv).astype(jnp.int32)
        e_idx = j * _LANES + iota
        for g in range(ngrp):
            in_pfx = (e_idx < base + g * _LANES).astype(jnp.int32)
            acc_pf[g] = acc_pf[g] + eq * in_pfx
    while len(accs_gt) > 1:
        accs_gt = [a + b for a, b in zip(accs_gt[::2], accs_gt[1::2])]
    n_gt = jnp.sum(accs_gt[0])
    rem = (keep - n_gt).astype(jnp.float32)

    for g in range(ngrp):
        d_idx = base + g * _LANES + iota
        dkey = _f32_key(plsc.load_gather(imp_v, [d_idx]))
        eqd = (dkey == thrv).astype(jnp.float32)
        tiepos = jnp.sum(acc_pf[g]).astype(jnp.float32) + (
            plsc.cumsum(eqd) - eqd)
        kept = (dkey > thrv) | ((dkey == thrv) & (tiepos < rem))
        out_v[pl.ds(g * _LANES, _LANES)] = jnp.where(kept, 1.0, 0.0)
    pltpu.sync_copy(out_v, mask_hbm.at[pl.ds(base, per)])


def _select_body(mask_ref, p_ref, x_ref, o_ref):
    x = x_ref[:, :]
    w0 = p_ref[0, 0]
    b0 = p_ref[0, 1]
    w1 = p_ref[0, 2]
    b1 = p_ref[0, 3]
    y = x * w0 + b0
    y = y * jax.nn.sigmoid(y)
    y = y * w1 + b1
    y = y * jax.nn.sigmoid(y)
    m = mask_ref[:, :] > 0.0
    o_ref[:, :] = jnp.where(m, y, x)


def _topk_mask(importance, keep):
    d_total = importance.shape[0]
    info = plsc.get_sparse_core_info()
    n_workers = info.num_cores * info.num_subcores
    per = d_total // n_workers
    mesh = plsc.VectorSubcoreMesh(core_axis_name="c", subcore_axis_name="s")
    sc_mask = pl.kernel(
        functools.partial(_sc_mask_body, keep, d_total, n_workers,
                          info.num_cores),
        mesh=mesh,
        compiler_params=pltpu.CompilerParams(needs_layout_passes=False),
        out_type=jax.ShapeDtypeStruct((d_total,), jnp.float32),
        scratch_types=[
            pltpu.VMEM((d_total,), jnp.float32),
            pltpu.VMEM((d_total,), jnp.uint32),
            pltpu.VMEM((per,), jnp.float32),
        ],
    )
    return sc_mask(importance).reshape(1, d_total)


def kernel(x, importance, w0, b0, w1, b1):
    b_sz, t_sz, d_sz = x.shape
    keep = max(1, int(d_sz * 0.25))
    rows_total = b_sz * t_sz
    xf = x.reshape(rows_total, d_sz)
    params = jnp.stack(
        [w0[0, 0], b0[0], w1[0, 0], b1[0]]).reshape(1, 4)

    mask = _topk_mask(importance, keep)

    rows = 512
    grid = (rows_total // rows,)
    out = pl.pallas_call(
        _select_body,
        grid=grid,
        in_specs=[
            pl.BlockSpec((1, d_sz), lambda i: (0, 0)),
            pl.BlockSpec(memory_space=pltpu.SMEM),
            pl.BlockSpec((rows, d_sz), lambda i: (i, 0)),
        ],
        out_specs=pl.BlockSpec((rows, d_sz), lambda i: (i, 0)),
        out_shape=jax.ShapeDtypeStruct((rows_total, d_sz), jnp.float32),
        compiler_params=pltpu.CompilerParams(
            dimension_semantics=("arbitrary",)),
    )(mask, params, xf)
    return out.reshape(b_sz, t_sz, d_sz)


# speedup vs baseline: 1.0566x; 1.0566x over previous
import functools

import jax
import jax.numpy as jnp
from jax import lax
from jax.experimental import pallas as pl
from jax.experimental.pallas import tpu as pltpu
from jax.experimental.pallas import tpu_sc as plsc

_LANES = 16


def _f32_key(f):
    u = plsc.bitcast(f, jnp.uint32)
    return jnp.where(f < 0.0, ~u, u | jnp.uint32(0x80000000))


def _sc_mask_body(keep, d_total, n_workers, n_cores,
                  imp_hbm, mask_hbm, imp_v, key_v, out_v):
    per = d_total // n_workers
    ngrp = per // _LANES
    nvec = d_total // _LANES
    wid = lax.axis_index("s") * n_cores + lax.axis_index("c")
    base = wid * per
    pltpu.sync_copy(imp_hbm, imp_v)
    iota = lax.broadcasted_iota(jnp.int32, (_LANES,), 0)

    for j in range(nvec):
        key_v[pl.ds(j * _LANES, _LANES)] = _f32_key(
            imp_v[pl.ds(j * _LANES, _LANES)])

    nacc = 8

    def bit_body(i, thr):
        t = thr | (jnp.uint32(1) << (jnp.uint32(31) - i.astype(jnp.uint32)))
        tv = jnp.broadcast_to(t, (_LANES,))
        accs = [jnp.zeros((_LANES,), jnp.int32) for _ in range(nacc)]
        for j in range(nvec):
            accs[j % nacc] = accs[j % nacc] + (
                key_v[pl.ds(j * _LANES, _LANES)] >= tv).astype(jnp.int32)
        while len(accs) > 1:
            accs = [a + b for a, b in zip(accs[::2], accs[1::2])]
        c = jnp.sum(accs[0])
        return jnp.where(c >= keep, t, thr)

    thr = lax.fori_loop(0, 32, bit_body, jnp.uint32(0))
    thrv = jnp.broadcast_to(thr, (_LANES,))

    accs_gt = [jnp.zeros((_LANES,), jnp.int32) for _ in range(nacc)]
    acc_pf = [jnp.zeros((_LANES,), jnp.int32) for _ in range(ngrp)]
    for j in range(nvec):
        kv = key_v[pl.ds(j * _LANES, _LANES)]
        accs_gt[j % nacc] = accs_gt[j % nacc] + (
            kv > thrv).astype(jnp.int32)
        eq = (kv == thrv).astype(jnp.int32)
        e_idx = j * _LANES + iota
        for g in range(ngrp):
            in_pfx = (e_idx < base + g * _LANES).astype(jnp.int32)
            acc_pf[g] = acc_pf[g] + eq * in_pfx
    while len(accs_gt) > 1:
        accs_gt = [a + b for a, b in zip(accs_gt[::2], accs_gt[1::2])]
    n_gt = jnp.sum(accs_gt[0])
    rem = (keep - n_gt).astype(jnp.float32)

    for g in range(ngrp):
        d_idx = base + g * _LANES + iota
        dkey = _f32_key(plsc.load_gather(imp_v, [d_idx]))
        eqd = (dkey == thrv).astype(jnp.float32)
        tiepos = jnp.sum(acc_pf[g]).astype(jnp.float32) + (
            plsc.cumsum(eqd) - eqd)
        kept = (dkey > thrv) | ((dkey == thrv) & (tiepos < rem))
        out_v[pl.ds(g * _LANES, _LANES)] = jnp.where(kept, 1.0, 0.0)
    pltpu.sync_copy(out_v, mask_hbm.at[pl.ds(base, per)])


def _select_body(mask_ref, p_ref, x_ref, o_ref):
    x = x_ref[:, :]
    w0 = p_ref[0, 0]
    b0 = p_ref[0, 1]
    w1 = p_ref[0, 2]
    b1 = p_ref[0, 3]
    y = x * w0 + b0
    y = y * jax.nn.sigmoid(y)
    y = y * w1 + b1
    y = y * jax.nn.sigmoid(y)
    m = mask_ref[:, :] > 0.0
    o_ref[:, :] = jnp.where(m, y, x)


def _topk_mask(importance, keep):
    d_total = importance.shape[0]
    info = plsc.get_sparse_core_info()
    n_workers = info.num_cores * info.num_subcores
    per = d_total // n_workers
    mesh = plsc.VectorSubcoreMesh(core_axis_name="c", subcore_axis_name="s")
    sc_mask = pl.kernel(
        functools.partial(_sc_mask_body, keep, d_total, n_workers,
                          info.num_cores),
        mesh=mesh,
        compiler_params=pltpu.CompilerParams(needs_layout_passes=False),
        out_type=jax.ShapeDtypeStruct((d_total,), jnp.float32),
        scratch_types=[
            pltpu.VMEM((d_total,), jnp.float32),
            pltpu.VMEM((d_total,), jnp.uint32),
            pltpu.VMEM((per,), jnp.float32),
        ],
    )
    return sc_mask(importance).reshape(1, d_total)


def kernel(x, importance, w0, b0, w1, b1):
    b_sz, t_sz, d_sz = x.shape
    keep = max(1, int(d_sz * 0.25))
    rows_total = b_sz * t_sz
    xf = x.reshape(rows_total, d_sz)
    params = jnp.stack(
        [w0[0, 0], b0[0], w1[0, 0], b1[0]]).reshape(1, 4)

    mask = _topk_mask(importance, keep)

    rows = 1024
    grid = (rows_total // rows,)
    out = pl.pallas_call(
        _select_body,
        grid=grid,
        in_specs=[
            pl.BlockSpec((1, d_sz), lambda i: (0, 0)),
            pl.BlockSpec(memory_space=pltpu.SMEM),
            pl.BlockSpec((rows, d_sz), lambda i: (i, 0)),
        ],
        out_specs=pl.BlockSpec((rows, d_sz), lambda i: (i, 0)),
        out_shape=jax.ShapeDtypeStruct((rows_total, d_sz), jnp.float32),
        compiler_params=pltpu.CompilerParams(
            dimension_semantics=("parallel",)),
    )(mask, params, xf)
    return out.reshape(b_sz, t_sz, d_sz)


# speedup vs baseline: 1.1168x; 1.0570x over previous
import functools

import jax
import jax.numpy as jnp
from jax import lax
from jax.experimental import pallas as pl
from jax.experimental.pallas import tpu as pltpu
from jax.experimental.pallas import tpu_sc as plsc

_LANES = 16


def _f32_key(f):
    u = plsc.bitcast(f, jnp.uint32)
    return jnp.where(f < 0.0, ~u, u | jnp.uint32(0x80000000))


def _sc_mask_body(keep, d_total, n_workers, n_cores,
                  imp_hbm, mask_hbm, imp_v, key_v, out_v):
    per = d_total // n_workers
    ngrp = per // _LANES
    nvec = d_total // _LANES
    wid = lax.axis_index("s") * n_cores + lax.axis_index("c")
    base = wid * per
    pltpu.sync_copy(imp_hbm, imp_v)
    iota = lax.broadcasted_iota(jnp.int32, (_LANES,), 0)

    for j in range(nvec):
        key_v[pl.ds(j * _LANES, _LANES)] = _f32_key(
            imp_v[pl.ds(j * _LANES, _LANES)])

    nacc = 8

    def bit_body(i, thr):
        t = thr | (jnp.uint32(1) << (jnp.uint32(31) - i.astype(jnp.uint32)))
        tv = jnp.broadcast_to(t, (_LANES,))
        accs = [jnp.zeros((_LANES,), jnp.int32) for _ in range(nacc)]
        for j in range(nvec):
            accs[j % nacc] = accs[j % nacc] + (
                key_v[pl.ds(j * _LANES, _LANES)] >= tv).astype(jnp.int32)
        while len(accs) > 1:
            accs = [a + b for a, b in zip(accs[::2], accs[1::2])]
        c = jnp.sum(accs[0])
        return jnp.where(c >= keep, t, thr)

    thr = lax.fori_loop(0, 32, bit_body, jnp.uint32(0))
    thrv = jnp.broadcast_to(thr, (_LANES,))

    accs_gt = [jnp.zeros((_LANES,), jnp.int32) for _ in range(nacc)]
    acc_pf = [jnp.zeros((_LANES,), jnp.int32) for _ in range(ngrp)]
    for j in range(nvec):
        kv = key_v[pl.ds(j * _LANES, _LANES)]
        accs_gt[j % nacc] = accs_gt[j % nacc] + (
            kv > thrv).astype(jnp.int32)
        eq = (kv == thrv).astype(jnp.int32)
        e_idx = j * _LANES + iota
        for g in range(ngrp):
            in_pfx = (e_idx < base + g * _LANES).astype(jnp.int32)
            acc_pf[g] = acc_pf[g] + eq * in_pfx
    while len(accs_gt) > 1:
        accs_gt = [a + b for a, b in zip(accs_gt[::2], accs_gt[1::2])]
    n_gt = jnp.sum(accs_gt[0])
    rem = (keep - n_gt).astype(jnp.float32)

    for g in range(ngrp):
        d_idx = base + g * _LANES + iota
        dkey = _f32_key(plsc.load_gather(imp_v, [d_idx]))
        eqd = (dkey == thrv).astype(jnp.float32)
        tiepos = jnp.sum(acc_pf[g]).astype(jnp.float32) + (
            plsc.cumsum(eqd) - eqd)
        kept = (dkey > thrv) | ((dkey == thrv) & (tiepos < rem))
        out_v[pl.ds(g * _LANES, _LANES)] = jnp.where(kept, 1.0, 0.0)
    pltpu.sync_copy(out_v, mask_hbm.at[pl.ds(base, per)])


def _select_body(mask_ref, p_ref, x_ref, o_ref):
    x = x_ref[:, :]
    a0 = p_ref[0, 0]
    c0 = p_ref[0, 1]
    a1 = p_ref[0, 2]
    c1 = p_ref[0, 3]
    h = x * a0 + c0
    s = h * jnp.tanh(h) + h
    h = s * a1 + c1
    r = h * jnp.tanh(h) + h
    m = mask_ref[:, :] > 0.0
    o_ref[:, :] = jnp.where(m, r, x)


def _topk_mask(importance, keep):
    d_total = importance.shape[0]
    info = plsc.get_sparse_core_info()
    n_workers = info.num_cores * info.num_subcores
    per = d_total // n_workers
    mesh = plsc.VectorSubcoreMesh(core_axis_name="c", subcore_axis_name="s")
    sc_mask = pl.kernel(
        functools.partial(_sc_mask_body, keep, d_total, n_workers,
                          info.num_cores),
        mesh=mesh,
        compiler_params=pltpu.CompilerParams(needs_layout_passes=False),
        out_type=jax.ShapeDtypeStruct((d_total,), jnp.float32),
        scratch_types=[
            pltpu.VMEM((d_total,), jnp.float32),
            pltpu.VMEM((d_total,), jnp.uint32),
            pltpu.VMEM((per,), jnp.float32),
        ],
    )
    return sc_mask(importance).reshape(1, d_total)


def kernel(x, importance, w0, b0, w1, b1):
    b_sz, t_sz, d_sz = x.shape
    keep = max(1, int(d_sz * 0.25))
    rows_total = b_sz * t_sz
    xf = x.reshape(rows_total, d_sz)
    params = (jnp.stack(
        [w0[0, 0], b0[0], w1[0, 0], b1[0]]) * 0.5).reshape(1, 4)

    mask = _topk_mask(importance, keep)

    rows = 1024
    grid = (rows_total // rows,)
    out = pl.pallas_call(
        _select_body,
        grid=grid,
        in_specs=[
            pl.BlockSpec((1, d_sz), lambda i: (0, 0)),
            pl.BlockSpec(memory_space=pltpu.SMEM),
            pl.BlockSpec((rows, d_sz), lambda i: (i, 0)),
        ],
        out_specs=pl.BlockSpec((rows, d_sz), lambda i: (i, 0)),
        out_shape=jax.ShapeDtypeStruct((rows_total, d_sz), jnp.float32),
        compiler_params=pltpu.CompilerParams(
            dimension_semantics=("parallel",)),
    )(mask, params, xf)
    return out.reshape(b_sz, t_sz, d_sz)
